# baseline (speedup 1.0000x reference)
import functools

import jax
import jax.numpy as jnp
from jax import lax
from jax.experimental import pallas as pl
from jax.experimental.pallas import tpu as pltpu
from jax.experimental.pallas import tpu_sc as plsc

BATCH = 16384
HIST = 200
N = BATCH * HIST

NUM_CORES = 2
NUM_SUBCORES = 16
NUM_WORKERS = NUM_CORES * NUM_SUBCORES
PER_WORKER = N // NUM_WORKERS
CHUNK = 10_240
NUM_CHUNKS = PER_WORKER // CHUNK
LANES = 16
SCALE = 2.0 / 3.0


def _sc_body(x_hbm, y_hbm, out_hbm, xbuf, ybuf, obuf):
    wid = lax.axis_index("s") * NUM_CORES + lax.axis_index("c")
    base = wid * PER_WORKER

    iota2 = lax.iota(jnp.int32, LANES) * 2

    def chunk_body(c, carry):
        start = base + c * CHUNK
        pltpu.sync_copy(x_hbm.at[pl.ds(start, CHUNK)], xbuf)
        pltpu.sync_copy(y_hbm.at[pl.ds(start, CHUNK)], ybuf)

        def vec_body(j, carry2):
            off = j * LANES
            vx = xbuf[pl.ds(off, LANES)]
            vy = ybuf[pl.ds(off, LANES)]
            fx = vx.astype(jnp.float32) * SCALE - 1.0
            fy = vy.astype(jnp.float32) * SCALE - 1.0
            idx = iota2 + 2 * off
            plsc.store_scatter(obuf, [idx], fx)
            plsc.store_scatter(obuf, [idx + 1], fy)
            return carry2

        lax.fori_loop(0, CHUNK // LANES, vec_body, 0, unroll=4)
        pltpu.sync_copy(obuf, out_hbm.at[pl.ds(2 * start, 2 * CHUNK)])
        return carry

    lax.fori_loop(0, NUM_CHUNKS, chunk_body, 0)


@jax.jit
def kernel(x_x, x_y):
    mesh = plsc.VectorSubcoreMesh(core_axis_name="c", subcore_axis_name="s")
    out_flat = pl.kernel(
        _sc_body,
        out_type=jax.ShapeDtypeStruct((2 * N,), jnp.float32),
        mesh=mesh,
        scratch_types=[
            pltpu.VMEM((CHUNK,), jnp.int32),
            pltpu.VMEM((CHUNK,), jnp.int32),
            pltpu.VMEM((2 * CHUNK,), jnp.float32),
        ],
        compiler_params=pltpu.CompilerParams(needs_layout_passes=False),
    )(x_x.reshape(-1), x_y.reshape(-1))
    return out_flat.reshape(BATCH, HIST, 2).astype(jnp.complex64)


# speedup vs baseline: 55.5862x; 55.5862x over previous
import jax
import jax.numpy as jnp
from jax import lax
from jax.experimental import pallas as pl
from jax.experimental.pallas import tpu as pltpu
from jax.experimental.pallas import tpu_sc as plsc

BATCH = 16384
HIST = 200

NUM_CORES = 2
NUM_SUBCORES = 16
NUM_WORKERS = NUM_CORES * NUM_SUBCORES
PER_WORKER = BATCH // NUM_WORKERS
BCH = 128
NUM_CHUNKS = PER_WORKER // BCH
LANES = 16
SCALE = 2.0 / 3.0


def _sc_body(x_hbm, y_hbm, out_hbm, xv, yv, outv):
    wid = lax.axis_index("s") * NUM_CORES + lax.axis_index("c")
    base = wid * PER_WORKER

    def chunk_body(c, carry):
        b0 = base + c * BCH
        pltpu.sync_copy(x_hbm.at[:, pl.ds(b0, BCH)], xv)
        pltpu.sync_copy(y_hbm.at[:, pl.ds(b0, BCH)], yv)

        def h_body(h, carry2):
            for bg in range(BCH // LANES):
                sl = pl.ds(bg * LANES, LANES)
                vx = xv[h, sl]
                vy = yv[h, sl]
                outv[0, h, sl] = vx.astype(jnp.float32) * SCALE - 1.0
                outv[1, h, sl] = vy.astype(jnp.float32) * SCALE - 1.0
            return carry2

        lax.fori_loop(0, HIST, h_body, 0)
        pltpu.sync_copy(outv, out_hbm.at[:, :, pl.ds(b0, BCH)])
        return carry

    lax.fori_loop(0, NUM_CHUNKS, chunk_body, 0)


@jax.jit
def kernel(x_x, x_y):
    mesh = plsc.VectorSubcoreMesh(core_axis_name="c", subcore_axis_name="s")
    val = pl.kernel(
        _sc_body,
        out_type=jax.ShapeDtypeStruct((2, HIST, BATCH), jnp.float32),
        mesh=mesh,
        scratch_types=[
            pltpu.VMEM((HIST, BCH), jnp.int32),
            pltpu.VMEM((HIST, BCH), jnp.int32),
            pltpu.VMEM((2, HIST, BCH), jnp.float32),
        ],
        compiler_params=pltpu.CompilerParams(needs_layout_passes=False),
    )(x_x.T, x_y.T)
    return jnp.transpose(val, (2, 1, 0)).astype(jnp.complex64)
